# Initial kernel scaffold; baseline (speedup 1.0000x reference)
import jax
import jax.numpy as jnp
from jax.experimental import pallas as pl


def kernel(ent_embs, rel_embs, m_freq, d_freq, y_freq, m_phi, d_phi, y_phi, m_amp, d_amp, y_amp, r1_freq, r2_freq, r1_phi, r2_phi, r1_amp, r2_amp, r1, r2, years, months, days, p2, p3, heads, rels, tails):
    raise NotImplementedError("write your pallas kernel here")


# speedup vs baseline: 1.5282x; 1.5282x over previous
import functools

import jax
import jax.numpy as jnp
from jax import lax
from jax.experimental import pallas as pl
from jax.experimental.pallas import tpu as pltpu
from jax.experimental.pallas import tpu_sc as plsc

_NC = 2
_NS = 16
_L = 16
_NW = _NC * _NS

_B = 16384
_SR = 48
_TD = 16
_BPW = _B // _NW
_CH = 128
_NCHUNK = _BPW // _CH

_TWO_PI = 6.283185307179586
_INV_2PI = 0.15915494309189535
_C3 = -1.0 / 6.0
_C5 = 1.0 / 120.0
_C7 = -1.0 / 5040.0
_C9 = 1.0 / 362880.0
_C11 = -1.0 / 39916800.0
_C13 = 1.0 / 6227020800.0


def _sin(x):
    k = (x * _INV_2PI + 0.5 * jnp.sign(x)).astype(jnp.int32).astype(jnp.float32)
    y = x - k * _TWO_PI
    y2 = y * y
    p = _C13 * y2 + _C11
    p = p * y2 + _C9
    p = p * y2 + _C7
    p = p * y2 + _C5
    p = p * y2 + _C3
    p = p * y2 + 1.0
    return y * p


def _body(ent_ref, rel_ref,
          mf_ref, df_ref, yf_ref, mp_ref, dp_ref, yp_ref, ma_ref, da_ref, ya_ref,
          r1f_ref, r2f_ref, r1p_ref, r2p_ref, r1a_ref, r2a_ref,
          heads_ref, tails_ref, rels_ref,
          yrs_ref, mon_ref, day_ref, r1_ref, r2_ref,
          out_ref,
          sem,
          ih_v, it_v, ir_v,
          yrs_v, mon_v, day_v, r1_v, r2_v,
          h_v, t_v, r_v,
          hyf_v, hmf_v, hdf_v, hyp_v, hmp_v, hdp_v, hya_v, hma_v, hda_v,
          tyf_v, tmf_v, tdf_v, typ_v, tmp_v, tdp_v, tya_v, tma_v, tda_v,
          g1f_v, g1p_v, g1a_v, g2f_v, g2p_v, g2a_v,
          out_v):
    wid = lax.axis_index("s") * _NC + lax.axis_index("c")
    lane = lax.iota(jnp.int32, _L)

    def chunk_body(c, carry):
        base = wid * _BPW + c * _CH
        pltpu.sync_copy(heads_ref.at[pl.ds(base, _CH)], ih_v)
        pltpu.sync_copy(tails_ref.at[pl.ds(base, _CH)], it_v)
        pltpu.sync_copy(rels_ref.at[pl.ds(base, _CH)], ir_v)
        rbase = wid * (_BPW // _L) + c * (_CH // _L)
        pltpu.sync_copy(yrs_ref.at[pl.ds(rbase, _CH // _L)], yrs_v)
        pltpu.sync_copy(mon_ref.at[pl.ds(rbase, _CH // _L)], mon_v)
        pltpu.sync_copy(day_ref.at[pl.ds(rbase, _CH // _L)], day_v)
        pltpu.sync_copy(r1_ref.at[pl.ds(rbase, _CH // _L)], r1_v)
        pltpu.sync_copy(r2_ref.at[pl.ds(rbase, _CH // _L)], r2_v)
        gathers = (
            (ent_ref, ih_v, h_v), (ent_ref, it_v, t_v), (rel_ref, ir_v, r_v),
            (yf_ref, ih_v, hyf_v), (mf_ref, ih_v, hmf_v), (df_ref, ih_v, hdf_v),
            (yp_ref, ih_v, hyp_v), (mp_ref, ih_v, hmp_v), (dp_ref, ih_v, hdp_v),
            (ya_ref, ih_v, hya_v), (ma_ref, ih_v, hma_v), (da_ref, ih_v, hda_v),
            (yf_ref, it_v, tyf_v), (mf_ref, it_v, tmf_v), (df_ref, it_v, tdf_v),
            (yp_ref, it_v, typ_v), (mp_ref, it_v, tmp_v), (dp_ref, it_v, tdp_v),
            (ya_ref, it_v, tya_v), (ma_ref, it_v, tma_v), (da_ref, it_v, tda_v),
            (r1f_ref, ir_v, g1f_v), (r1p_ref, ir_v, g1p_v), (r1a_ref, ir_v, g1a_v),
            (r2f_ref, ir_v, g2f_v), (r2p_ref, ir_v, g2p_v), (r2a_ref, ir_v, g2a_v),
        )
        descs = [pltpu.async_copy(tbl.at[idx], dst, sem)
                 for tbl, idx, dst in gathers]
        for d in descs:
            d.wait()

        def group_body(g, carry2):
            e = g * _L + lane
            yrs = yrs_v[g, :]
            mon = mon_v[g, :]
            day = day_v[g, :]
            r1s = r1_v[g, :]
            r2s = r2_v[g, :]

            def ent_body(d, acc):
                dv = jnp.full((_L,), d, jnp.int32)
                h = plsc.load_gather(h_v, [e, dv])
                r = plsc.load_gather(r_v, [e, dv])
                t = plsc.load_gather(t_v, [e, dv])
                return acc + h * r * t

            acc = lax.fori_loop(0, _SR, ent_body, jnp.zeros((_L,), jnp.float32))

            def time_body(d, acc):
                dv = jnp.full((_L,), d, jnp.int32)

                def term(a_ref, f_ref, p_ref, idx, t):
                    a = plsc.load_gather(a_ref, [e, dv])
                    f = plsc.load_gather(f_ref, [e, dv])
                    p = plsc.load_gather(p_ref, [e, dv])
                    return a * _sin(f * t + p)

                h_t = (term(hya_v, hyf_v, hyp_v, e, yrs)
                       + term(hma_v, hmf_v, hmp_v, e, mon)
                       + term(hda_v, hdf_v, hdp_v, e, day))
                t_t = (term(tya_v, tyf_v, typ_v, e, yrs)
                       + term(tma_v, tmf_v, tmp_v, e, mon)
                       + term(tda_v, tdf_v, tdp_v, e, day))
                r_r = (term(g1a_v, g1f_v, g1p_v, e, r1s)
                       + term(g2a_v, g2f_v, g2p_v, e, r2s))
                return acc + h_t * r_r * t_t

            acc = lax.fori_loop(0, _TD, time_body, acc)
            out_v[c * (_CH // _L) + g, :] = acc
            return carry2

        return lax.fori_loop(0, _CH // _L, group_body, carry)

    lax.fori_loop(0, _NCHUNK, chunk_body, jnp.int32(0))
    pltpu.sync_copy(out_v, out_ref.at[pl.ds(wid * (_BPW // _L), _BPW // _L)])


@functools.lru_cache(maxsize=1)
def _de_score():
    return pl.kernel(
        _body,
        out_type=jax.ShapeDtypeStruct((_B // _L, _L), jnp.float32),
        mesh=plsc.VectorSubcoreMesh(core_axis_name="c", subcore_axis_name="s",
                                    num_cores=_NC, num_subcores=_NS),
        compiler_params=pltpu.CompilerParams(needs_layout_passes=False,
                                             use_tc_tiling_on_sc=False),
        scratch_types=(
            [pltpu.SemaphoreType.DMA]
            + [pltpu.VMEM((_CH,), jnp.int32)] * 3
            + [pltpu.VMEM((_CH // _L, _L), jnp.float32)] * 5
            + [pltpu.VMEM((_CH, _SR), jnp.float32)] * 3
            + [pltpu.VMEM((_CH, _TD), jnp.float32)] * 24
            + [pltpu.VMEM((_BPW // _L, _L), jnp.float32)]
        ),
    )


def kernel(ent_embs, rel_embs, m_freq, d_freq, y_freq, m_phi, d_phi, y_phi,
           m_amp, d_amp, y_amp, r1_freq, r2_freq, r1_phi, r2_phi, r1_amp,
           r2_amp, r1, r2, years, months, days, p2, p3, heads, rels, tails):
    out = _de_score()(ent_embs, rel_embs,
                      m_freq, d_freq, y_freq, m_phi, d_phi, y_phi,
                      m_amp, d_amp, y_amp,
                      r1_freq, r2_freq, r1_phi, r2_phi, r1_amp, r2_amp,
                      heads, tails, rels,
                      years.reshape(-1, _L), months.reshape(-1, _L),
                      days.reshape(-1, _L), r1.reshape(-1, _L),
                      r2.reshape(-1, _L))
    return out.reshape(-1)


# speedup vs baseline: 1.8042x; 1.1806x over previous
import functools

import jax
import jax.numpy as jnp
from jax import lax
from jax.experimental import pallas as pl
from jax.experimental.pallas import tpu as pltpu
from jax.experimental.pallas import tpu_sc as plsc

_NC = 2
_NS = 16
_L = 16
_NW = _NC * _NS

_B = 16384
_SR = 48
_TD = 16
_BPW = _B // _NW
_CH = 128
_NCHUNK = _BPW // _CH

_TWO_PI = 6.283185307179586
_INV_2PI = 0.15915494309189535
_C3 = -1.0 / 6.0
_C5 = 1.0 / 120.0
_C7 = -1.0 / 5040.0
_C9 = 1.0 / 362880.0
_C11 = -1.0 / 39916800.0
_C13 = 1.0 / 6227020800.0


def _sin(x):
    k = (x * _INV_2PI + 0.5 * jnp.sign(x)).astype(jnp.int32).astype(jnp.float32)
    y = x - k * _TWO_PI
    y2 = y * y
    p = _C13 * y2 + _C11
    p = p * y2 + _C9
    p = p * y2 + _C7
    p = p * y2 + _C5
    p = p * y2 + _C3
    p = p * y2 + 1.0
    return y * p


def _body(ent_ref, rel_ref,
          mf_ref, df_ref, yf_ref, mp_ref, dp_ref, yp_ref, ma_ref, da_ref, ya_ref,
          r1f_ref, r2f_ref, r1p_ref, r2p_ref, r1a_ref, r2a_ref,
          heads_ref, tails_ref, rels_ref,
          yrs_ref, mon_ref, day_ref, r1_ref, r2_ref,
          out_ref,
          sem,
          ih_v, it_v, ir_v,
          yrs_v, mon_v, day_v, r1_v, r2_v,
          h_v, t_v, r_v,
          hyf_v, hmf_v, hdf_v, hyp_v, hmp_v, hdp_v, hya_v, hma_v, hda_v,
          tyf_v, tmf_v, tdf_v, typ_v, tmp_v, tdp_v, tya_v, tma_v, tda_v,
          g1f_v, g1p_v, g1a_v, g2f_v, g2p_v, g2a_v,
          out_v):
    wid = lax.axis_index("s") * _NC + lax.axis_index("c")
    lane = lax.iota(jnp.int32, _L)

    def chunk_body(c, carry):
        base = wid * _BPW + c * _CH
        pltpu.sync_copy(heads_ref.at[pl.ds(base, _CH)], ih_v)
        pltpu.sync_copy(tails_ref.at[pl.ds(base, _CH)], it_v)
        pltpu.sync_copy(rels_ref.at[pl.ds(base, _CH)], ir_v)
        rbase = wid * (_BPW // _L) + c * (_CH // _L)
        pltpu.sync_copy(yrs_ref.at[pl.ds(rbase, _CH // _L)], yrs_v)
        pltpu.sync_copy(mon_ref.at[pl.ds(rbase, _CH // _L)], mon_v)
        pltpu.sync_copy(day_ref.at[pl.ds(rbase, _CH // _L)], day_v)
        pltpu.sync_copy(r1_ref.at[pl.ds(rbase, _CH // _L)], r1_v)
        pltpu.sync_copy(r2_ref.at[pl.ds(rbase, _CH // _L)], r2_v)
        gathers = (
            (ent_ref, ih_v, h_v), (ent_ref, it_v, t_v), (rel_ref, ir_v, r_v),
            (yf_ref, ih_v, hyf_v), (mf_ref, ih_v, hmf_v), (df_ref, ih_v, hdf_v),
            (yp_ref, ih_v, hyp_v), (mp_ref, ih_v, hmp_v), (dp_ref, ih_v, hdp_v),
            (ya_ref, ih_v, hya_v), (ma_ref, ih_v, hma_v), (da_ref, ih_v, hda_v),
            (yf_ref, it_v, tyf_v), (mf_ref, it_v, tmf_v), (df_ref, it_v, tdf_v),
            (yp_ref, it_v, typ_v), (mp_ref, it_v, tmp_v), (dp_ref, it_v, tdp_v),
            (ya_ref, it_v, tya_v), (ma_ref, it_v, tma_v), (da_ref, it_v, tda_v),
            (r1f_ref, ir_v, g1f_v), (r1p_ref, ir_v, g1p_v), (r1a_ref, ir_v, g1a_v),
            (r2f_ref, ir_v, g2f_v), (r2p_ref, ir_v, g2p_v), (r2a_ref, ir_v, g2a_v),
        )
        descs = [pltpu.async_copy(tbl.at[idx], dst, sem)
                 for tbl, idx, dst in gathers]
        for d in descs:
            d.wait()

        def group_body(g, carry2):
            e = g * _L + lane
            yrs = yrs_v[g, :]
            mon = mon_v[g, :]
            day = day_v[g, :]
            r1s = r1_v[g, :]
            r2s = r2_v[g, :]

            def ent_body(d, acc):
                dv = jnp.full((_L,), d, jnp.int32)
                h = plsc.load_gather(h_v, [e, dv])
                r = plsc.load_gather(r_v, [e, dv])
                t = plsc.load_gather(t_v, [e, dv])
                return acc + h * r * t

            acc = lax.fori_loop(0, _SR, ent_body, jnp.zeros((_L,), jnp.float32))

            def time_body(d, acc):
                dv = jnp.full((_L,), d, jnp.int32)

                def term(a_ref, f_ref, p_ref, idx, t):
                    a = plsc.load_gather(a_ref, [e, dv])
                    f = plsc.load_gather(f_ref, [e, dv])
                    p = plsc.load_gather(p_ref, [e, dv])
                    return a * _sin(f * t + p)

                h_t = (term(hya_v, hyf_v, hyp_v, e, yrs)
                       + term(hma_v, hmf_v, hmp_v, e, mon)
                       + term(hda_v, hdf_v, hdp_v, e, day))
                t_t = (term(tya_v, tyf_v, typ_v, e, yrs)
                       + term(tma_v, tmf_v, tmp_v, e, mon)
                       + term(tda_v, tdf_v, tdp_v, e, day))
                r_r = (term(g1a_v, g1f_v, g1p_v, e, r1s)
                       + term(g2a_v, g2f_v, g2p_v, e, r2s))
                return acc + h_t * r_r * t_t

            acc = lax.fori_loop(0, _TD, time_body, acc)
            out_v[c * (_CH // _L) + g, :] = acc
            return carry2

        return lax.fori_loop(0, _CH // _L, group_body, carry)

    lax.fori_loop(0, _NCHUNK, chunk_body, jnp.int32(0))
    pltpu.sync_copy(out_v, out_ref.at[pl.ds(wid * (_BPW // _L), _BPW // _L)])


@functools.lru_cache(maxsize=1)
def _de_score():
    return pl.kernel(
        _body,
        out_type=jax.ShapeDtypeStruct((_B // _L, _L), jnp.float32),
        mesh=plsc.VectorSubcoreMesh(core_axis_name="c", subcore_axis_name="s",
                                    num_cores=_NC, num_subcores=_NS),
        compiler_params=pltpu.CompilerParams(needs_layout_passes=False,
                                             use_tc_tiling_on_sc=False),
        scratch_types=(
            [pltpu.SemaphoreType.DMA]
            + [pltpu.VMEM((_CH,), jnp.int32)] * 3
            + [pltpu.VMEM((_CH // _L, _L), jnp.float32)] * 5
            + [pltpu.VMEM((_CH, _SR), jnp.float32)] * 3
            + [pltpu.VMEM((_CH, _TD), jnp.float32)] * 24
            + [pltpu.VMEM((_BPW // _L, _L), jnp.float32)]
        ),
    )


def kernel(ent_embs, rel_embs, m_freq, d_freq, y_freq, m_phi, d_phi, y_phi,
           m_amp, d_amp, y_amp, r1_freq, r2_freq, r1_phi, r2_phi, r1_amp,
           r2_amp, r1, r2, years, months, days, p2, p3, heads, rels, tails):
    from jax.experimental.layout import Layout, with_layout_constraint
    _rm = Layout(major_to_minor=(0, 1))
    (m_freq, d_freq, y_freq, m_phi, d_phi, y_phi,
     m_amp, d_amp, y_amp, r1_freq, r2_freq, r1_phi, r2_phi, r1_amp,
     r2_amp) = [
        with_layout_constraint(t, _rm)
        for t in (m_freq, d_freq, y_freq, m_phi, d_phi,
                  y_phi, m_amp, d_amp, y_amp, r1_freq, r2_freq, r1_phi,
                  r2_phi, r1_amp, r2_amp)]
    out = _de_score()(ent_embs, rel_embs,
                      m_freq, d_freq, y_freq, m_phi, d_phi, y_phi,
                      m_amp, d_amp, y_amp,
                      r1_freq, r2_freq, r1_phi, r2_phi, r1_amp, r2_amp,
                      heads, tails, rels,
                      years.reshape(-1, _L), months.reshape(-1, _L),
                      days.reshape(-1, _L), r1.reshape(-1, _L),
                      r2.reshape(-1, _L))
    return out.reshape(-1)
